# baseline (speedup 1.0000x reference)
import jax
import jax.numpy as jnp
from jax.experimental import pallas as pl
from jax.experimental.pallas import tpu as pltpu

_BM = 256


def _dot(a, b):
    return jax.lax.dot_general(
        a, b,
        dimension_numbers=(((1,), (0,)), ((), ())),
        preferred_element_type=jnp.float32,
    )


def _matmul_block(mat_a_ref, mat_b_ref, inp_ref, out_ref):
    out_ref[:_BM, :] = _dot(mat_a_ref[...], inp_ref[...])
    out_ref[_BM:, :] = _dot(mat_b_ref[...], inp_ref[...])


def kernel(inp, matrix):
    B, C, S = inp.shape
    M, K = matrix.shape
    inp_flat = inp.reshape(B * C, S)

    out_flat = pl.pallas_call(
        _matmul_block,
        grid=(M // (2 * _BM),),
        in_specs=[
            pl.BlockSpec((_BM, K), lambda i: (2 * i, 0)),
            pl.BlockSpec((_BM, K), lambda i: (2 * i + 1, 0)),
            pl.BlockSpec((B * C, S), lambda i: (0, 0)),
        ],
        out_specs=pl.BlockSpec((2 * _BM, S), lambda i: (i, 0)),
        out_shape=jax.ShapeDtypeStruct((M, S), jnp.float32),
        compiler_params=pltpu.CompilerParams(
            dimension_semantics=("arbitrary",),
        ),
    )(matrix, matrix, inp_flat)

    return out_flat.reshape(B, C, S)
